# Initial kernel scaffold; baseline (speedup 1.0000x reference)
import jax
import jax.numpy as jnp
from jax.experimental import pallas as pl


def kernel(x, W_enc, W_dec, bias):
    raise NotImplementedError("write your pallas kernel here")


# speedup vs baseline: 1.0002x; 1.0002x over previous
import functools

import jax
import jax.numpy as jnp
from jax.experimental import pallas as pl
from jax.experimental.pallas import tpu as pltpu

INPUT_DIM = 2048
HIDDEN_DIM = 16384
TOPK = 64
BATCH = 2048

BH = 512


def _encode_body(x_ref, w_ref, b_ref, out_ref):
    xm = (x_ref[...] - b_ref[...][None, :]).astype(jnp.bfloat16)
    out_ref[...] = jax.lax.dot_general(
        xm, w_ref[...].astype(jnp.bfloat16),
        dimension_numbers=(((1,), (1,)), ((), ())),
        preferred_element_type=jnp.float32,
    )


def _encode(x, W_enc, bias):
    grid = (HIDDEN_DIM // BH,)
    return pl.pallas_call(
        _encode_body,
        grid=grid,
        in_specs=[
            pl.BlockSpec((BATCH, INPUT_DIM), lambda h: (0, 0)),
            pl.BlockSpec((BH, INPUT_DIM), lambda h: (h, 0)),
            pl.BlockSpec((INPUT_DIM,), lambda h: (0,)),
        ],
        out_specs=pl.BlockSpec((BATCH, BH), lambda h: (0, h)),
        out_shape=jax.ShapeDtypeStruct((BATCH, HIDDEN_DIM), jnp.float32),
    )(x, W_enc, bias)


def kernel(x, W_enc, W_dec, bias):
    pre = _encode(x, W_enc, bias)
    vals, idx = jax.lax.top_k(pre, TOPK)
    vals = jax.nn.relu(vals)
    rows = jnp.arange(pre.shape[0])[:, None]
    z = jnp.zeros_like(pre).at[rows, idx].set(vals)
    recon = jnp.matmul(z, jnp.transpose(W_dec)) + bias
    return recon
